# baseline (speedup 1.0000x reference)
import functools

import jax
import jax.numpy as jnp
from jax import lax
from jax.experimental import pallas as pl
from jax.experimental.pallas import tpu as pltpu
from jax.experimental.pallas import tpu_sc as plsc

NC = 2
NS = 16
NW = NC * NS
DIM = 32
BATCH = 16384
VOCAB = 1000000
B_PER_W = BATCH // NW
CHUNK = 128
NCHUNK = B_PER_W // CHUNK
BLK = 49152
GRID = (VOCAB + BLK - 1) // BLK


def _p1_body(ent_ref, pred_ref, we_ref, wp_ref, ys_ref, yo_ref, yp_ref):
    eo = jnp.dot(we_ref[...], ent_ref[...], preferred_element_type=jnp.float32)
    ys_ref[...] = eo[0]
    yo_ref[...] = eo[1]
    yp_ref[...] = jnp.dot(wp_ref[...], pred_ref[...],
                          preferred_element_type=jnp.float32)[0]


def _sc_body(ids_hbm, ys_hbm, yp_hbm, yo_hbm, wb_hbm, out_hbm,
             sidx, pidx, oidx, gs, gp, go, scores, wv, sem):
    wid = lax.axis_index("s") * NC + lax.axis_index("c")
    base = wid * B_PER_W

    pltpu.sync_copy(ids_hbm.at[0, wid], sidx)
    pltpu.sync_copy(ids_hbm.at[1, wid], pidx)
    pltpu.sync_copy(ids_hbm.at[2, wid], oidx)
    pltpu.sync_copy(wb_hbm, wv)

    descs = []
    for k in range(NCHUNK):
        dst = pl.ds(k * CHUNK, CHUNK)
        descs.append(pltpu.async_copy(ys_hbm.at[sidx.at[k]], gs.at[dst], sem))
        descs.append(pltpu.async_copy(yp_hbm.at[pidx.at[k]], gp.at[dst], sem))
        descs.append(pltpu.async_copy(yo_hbm.at[oidx.at[k]], go.at[dst], sem))
    for d in descs:
        d.wait()

    bias = wv[pl.ds(0, 16)][0]
    for v in range(B_PER_W // 16):
        sl = pl.ds(v * 16, 16)
        scores[sl] = gs[sl] + gp[sl] + go[sl] + bias

    pltpu.sync_copy(scores, out_hbm.at[pl.ds(base, B_PER_W)])


@jax.jit
def _triple_score(ids_r, ent_t, pred_t, we, wp, wb):
    ys, yo, yp = pl.pallas_call(
        _p1_body,
        grid=(GRID,),
        in_specs=[
            pl.BlockSpec((DIM, BLK), lambda i: (0, i)),
            pl.BlockSpec((DIM, BLK), lambda i: (0, i)),
            pl.BlockSpec((2, DIM), lambda i: (0, 0)),
            pl.BlockSpec((1, DIM), lambda i: (0, 0)),
        ],
        out_specs=[
            pl.BlockSpec((BLK,), lambda i: (i,)),
            pl.BlockSpec((BLK,), lambda i: (i,)),
            pl.BlockSpec((BLK,), lambda i: (i,)),
        ],
        out_shape=[
            jax.ShapeDtypeStruct((VOCAB,), jnp.float32),
            jax.ShapeDtypeStruct((VOCAB,), jnp.float32),
            jax.ShapeDtypeStruct((VOCAB,), jnp.float32),
        ],
    )(ent_t, pred_t, we, wp)

    mesh = plsc.VectorSubcoreMesh(core_axis_name="c", subcore_axis_name="s")
    f = functools.partial(
        pl.kernel,
        out_type=jax.ShapeDtypeStruct((BATCH,), jnp.float32),
        mesh=mesh,
        scratch_types=[
            pltpu.VMEM((NCHUNK, CHUNK), jnp.int32),
            pltpu.VMEM((NCHUNK, CHUNK), jnp.int32),
            pltpu.VMEM((NCHUNK, CHUNK), jnp.int32),
            pltpu.VMEM((B_PER_W,), jnp.float32),
            pltpu.VMEM((B_PER_W,), jnp.float32),
            pltpu.VMEM((B_PER_W,), jnp.float32),
            pltpu.VMEM((B_PER_W,), jnp.float32),
            pltpu.VMEM((16,), jnp.float32),
            pltpu.SemaphoreType.DMA,
        ],
        compiler_params=pltpu.CompilerParams(
            needs_layout_passes=False, use_tc_tiling_on_sc=False),
    )(_sc_body)
    return f(ids_r, ys, yp, yo, wb)


def kernel(triple_ids, entity_emb, pred_emb, W, b):
    if triple_ids.ndim == 1:
        triple_ids = triple_ids[None, :]
    ids_r = triple_ids.T.astype(jnp.int32).reshape(3, NW, NCHUNK, CHUNK)
    w3 = W.reshape(3, DIM)
    we = jnp.stack([w3[0], w3[2]])
    wp = w3[1].reshape(1, DIM)
    wb = jnp.broadcast_to(b.reshape(1), (16,)).astype(jnp.float32)
    return _triple_score(ids_r, entity_emb.T, pred_emb.T, we, wp, wb)


# speedup vs baseline: 1.0063x; 1.0063x over previous
import functools

import jax
import jax.numpy as jnp
from jax import lax
from jax.experimental import pallas as pl
from jax.experimental.pallas import tpu as pltpu
from jax.experimental.pallas import tpu_sc as plsc

NC = 2
NS = 16
NW = NC * NS
DIM = 32
BATCH = 16384
VOCAB = 1000000
B_PER_W = BATCH // NW
CHUNK = 128
NCHUNK = B_PER_W // CHUNK
BLK = 24576
GRID = (VOCAB + BLK - 1) // BLK


def _p1_body(ent_ref, pred_ref, we_ref, wp_ref, ys_ref, yo_ref, yp_ref):
    eo = jnp.dot(we_ref[...], ent_ref[...], preferred_element_type=jnp.float32)
    ys_ref[...] = eo[0]
    yo_ref[...] = eo[1]
    yp_ref[...] = jnp.dot(wp_ref[...], pred_ref[...],
                          preferred_element_type=jnp.float32)[0]


def _sc_body(ids_hbm, ys_hbm, yp_hbm, yo_hbm, wb_hbm, out_hbm,
             sidx, pidx, oidx, gs, gp, go, scores, wv, sem):
    wid = lax.axis_index("s") * NC + lax.axis_index("c")
    base = wid * B_PER_W

    pltpu.sync_copy(ids_hbm.at[0, wid], sidx)
    pltpu.sync_copy(ids_hbm.at[1, wid], pidx)
    pltpu.sync_copy(ids_hbm.at[2, wid], oidx)
    pltpu.sync_copy(wb_hbm, wv)

    descs = []
    for k in range(NCHUNK):
        dst = pl.ds(k * CHUNK, CHUNK)
        descs.append(pltpu.async_copy(ys_hbm.at[sidx.at[k]], gs.at[dst], sem))
        descs.append(pltpu.async_copy(yp_hbm.at[pidx.at[k]], gp.at[dst], sem))
        descs.append(pltpu.async_copy(yo_hbm.at[oidx.at[k]], go.at[dst], sem))
    for d in descs:
        d.wait()

    bias = wv[pl.ds(0, 16)][0]
    for v in range(B_PER_W // 16):
        sl = pl.ds(v * 16, 16)
        scores[sl] = gs[sl] + gp[sl] + go[sl] + bias

    pltpu.sync_copy(scores, out_hbm.at[pl.ds(base, B_PER_W)])


@jax.jit
def _triple_score(ids_r, ent_t, pred_t, we, wp, wb):
    ys, yo, yp = pl.pallas_call(
        _p1_body,
        grid=(GRID,),
        in_specs=[
            pl.BlockSpec((DIM, BLK), lambda i: (0, i)),
            pl.BlockSpec((DIM, BLK), lambda i: (0, i)),
            pl.BlockSpec((2, DIM), lambda i: (0, 0)),
            pl.BlockSpec((1, DIM), lambda i: (0, 0)),
        ],
        out_specs=[
            pl.BlockSpec((BLK,), lambda i: (i,)),
            pl.BlockSpec((BLK,), lambda i: (i,)),
            pl.BlockSpec((BLK,), lambda i: (i,)),
        ],
        out_shape=[
            jax.ShapeDtypeStruct((VOCAB,), jnp.float32),
            jax.ShapeDtypeStruct((VOCAB,), jnp.float32),
            jax.ShapeDtypeStruct((VOCAB,), jnp.float32),
        ],
    )(ent_t, pred_t, we, wp)

    mesh = plsc.VectorSubcoreMesh(core_axis_name="c", subcore_axis_name="s")
    f = functools.partial(
        pl.kernel,
        out_type=jax.ShapeDtypeStruct((BATCH,), jnp.float32),
        mesh=mesh,
        scratch_types=[
            pltpu.VMEM((NCHUNK, CHUNK), jnp.int32),
            pltpu.VMEM((NCHUNK, CHUNK), jnp.int32),
            pltpu.VMEM((NCHUNK, CHUNK), jnp.int32),
            pltpu.VMEM((B_PER_W,), jnp.float32),
            pltpu.VMEM((B_PER_W,), jnp.float32),
            pltpu.VMEM((B_PER_W,), jnp.float32),
            pltpu.VMEM((B_PER_W,), jnp.float32),
            pltpu.VMEM((16,), jnp.float32),
            pltpu.SemaphoreType.DMA,
        ],
        compiler_params=pltpu.CompilerParams(
            needs_layout_passes=False, use_tc_tiling_on_sc=False),
    )(_sc_body)
    return f(ids_r, ys, yp, yo, wb)


def kernel(triple_ids, entity_emb, pred_emb, W, b):
    if triple_ids.ndim == 1:
        triple_ids = triple_ids[None, :]
    ids_r = triple_ids.T.astype(jnp.int32).reshape(3, NW, NCHUNK, CHUNK)
    w3 = W.reshape(3, DIM)
    we = jnp.stack([w3[0], w3[2]])
    wp = w3[1].reshape(1, DIM)
    wb = jnp.broadcast_to(b.reshape(1), (16,)).astype(jnp.float32)
    return _triple_score(ids_r, entity_emb.T, pred_emb.T, we, wp, wb)
